# baseline (speedup 1.0000x reference)
import jax
import jax.numpy as jnp
from jax import lax
from jax.experimental import pallas as pl
from jax.experimental.pallas import tpu as pltpu
from jax.experimental.pallas import tpu_sc as plsc

VOCAB_SZ = 100000
EMBED = 128
NCLS = 64
BATCH_SZ = 4096
SEQ_LEN = 200
LANES = 16
NCHUNK = NCLS // LANES
NBGRP = BPW_GROUPS = 8

NC, NS = 2, 16
NW = NC * NS
BPW = BATCH_SZ // NW
NSLOT = 8
NGRP = SEQ_LEN // NSLOT

VBLK = 1000


def _proj_body(emb_ref, w_ref, out_ref):
    out_ref[...] = lax.dot_general(
        emb_ref[...].astype(jnp.bfloat16), w_ref[...].astype(jnp.bfloat16),
        dimension_numbers=(((1,), (1,)), ((), ())),
        preferred_element_type=jnp.float32)


def _pool_body(ids_hbm, proj_hbm, fcb_hbm, out_hbm, ids_v, ring_v, buf_v,
               bias_v, *sems):
    wid = lax.axis_index("s") * NC + lax.axis_index("c")
    pltpu.sync_copy(ids_hbm.at[pl.ds(wid * BPW * SEQ_LEN, BPW * SEQ_LEN)],
                    ids_v)
    pltpu.sync_copy(fcb_hbm, bias_v)

    rows = [(lax.iota(jnp.int32, LANES) + r * LANES) * SEQ_LEN
            for r in range(NBGRP)]

    def build_ring_row(k, j):
        for r in range(NBGRP):
            ring_v[k, pl.ds(r * LANES, LANES)] = plsc.load_gather(
                ids_v, [rows[r] + j])

    for k in range(NSLOT):
        build_ring_row(k, jnp.int32(k))
        pltpu.async_copy(proj_hbm.at[ring_v.at[k]], buf_v.at[k], sems[k])

    def grp_body(g, carry):
        for k in range(NSLOT):
            pltpu.make_async_copy(proj_hbm.at[pl.ds(0, BPW)], buf_v.at[k],
                                  sems[k]).wait()
            build_ring_row(k, g * NSLOT + k)
            pltpu.async_copy(proj_hbm.at[ring_v.at[k]], buf_v.at[k], sems[k],
                             add=True)
        return carry

    lax.fori_loop(1, NGRP, grp_body, 0)
    for k in range(NSLOT):
        pltpu.make_async_copy(proj_hbm.at[pl.ds(0, BPW)], buf_v.at[k],
                              sems[k]).wait()

    def comb_body(b, carry):
        for c in range(NCHUNK):
            a = bias_v[pl.ds(c * LANES, LANES)] * float(SEQ_LEN)
            for k in range(NSLOT):
                a = a + buf_v[k, b, pl.ds(c * LANES, LANES)]
            buf_v[0, b, pl.ds(c * LANES, LANES)] = a
        return carry

    lax.fori_loop(0, BPW, comb_body, 0)
    pltpu.sync_copy(buf_v.at[0], out_hbm.at[pl.ds(wid * BPW, BPW)])


def kernel(input_ids, emb_table, fc_w, fc_b):
    proj = pl.pallas_call(
        _proj_body,
        grid=(VOCAB_SZ // VBLK,),
        in_specs=[
            pl.BlockSpec((VBLK, EMBED), lambda i: (i, 0)),
            pl.BlockSpec((NCLS, EMBED), lambda i: (0, 0)),
        ],
        out_specs=pl.BlockSpec((VBLK, NCLS), lambda i: (i, 0)),
        out_shape=jax.ShapeDtypeStruct((VOCAB_SZ, NCLS), jnp.float32),
    )(emb_table, fc_w)

    pool = pl.kernel(
        _pool_body,
        out_type=jax.ShapeDtypeStruct((BATCH_SZ, NCLS), jnp.float32),
        mesh=plsc.VectorSubcoreMesh(core_axis_name="c", subcore_axis_name="s"),
        compiler_params=pltpu.CompilerParams(use_tc_tiling_on_sc=False,
                                             needs_layout_passes=False),
        scratch_types=[
            pltpu.VMEM((BPW * SEQ_LEN,), jnp.int32),
            pltpu.VMEM((NSLOT, BPW), jnp.int32),
            pltpu.VMEM((NSLOT, BPW, NCLS), jnp.float32),
            pltpu.VMEM((NCLS,), jnp.float32),
        ] + [pltpu.SemaphoreType.DMA] * NSLOT,
    )
    return pool(input_ids.reshape(-1), proj, fc_b)


# speedup vs baseline: 1.5768x; 1.5768x over previous
import jax
import jax.numpy as jnp
from jax import lax
from jax.experimental import pallas as pl
from jax.experimental.pallas import tpu as pltpu
from jax.experimental.pallas import tpu_sc as plsc

VOCAB_SZ = 100000
EMBED = 128
NCLS = 64
BATCH_SZ = 4096
SEQ_LEN = 200
LANES = 16
NCHUNK = NCLS // LANES
NBGRP = BPW_GROUPS = 8

NC, NS = 2, 16
NW = NC * NS
BPW = BATCH_SZ // NW
NSLOT = 8
NGRP = SEQ_LEN // NSLOT

VBLK = 2000
HALF = VOCAB_SZ // 2


def _proj_body(lo_ref, hi_ref, w2_ref, out_ref):
    pair = jnp.concatenate([lo_ref[...], hi_ref[...]], axis=1)
    out_ref[...] = lax.dot_general(
        pair.astype(jnp.bfloat16), w2_ref[...].astype(jnp.bfloat16),
        dimension_numbers=(((1,), (0,)), ((), ())),
        preferred_element_type=jnp.float32)


def _pool_body(ids_hbm, proj_hbm, fcb_hbm, out_hbm, ids_v, ring_v, buf_v,
               bias_v, *sems):
    wid = lax.axis_index("s") * NC + lax.axis_index("c")
    pltpu.sync_copy(ids_hbm.at[pl.ds(wid * BPW * SEQ_LEN, BPW * SEQ_LEN)],
                    ids_v)
    pltpu.sync_copy(fcb_hbm, bias_v)

    rows = [(lax.iota(jnp.int32, LANES) + r * LANES) * SEQ_LEN
            for r in range(NBGRP)]

    def build_ring_row(k, j):
        for r in range(NBGRP):
            t = plsc.load_gather(ids_v, [rows[r] + j])
            t2 = t * 2 - jnp.where(t >= HALF, VOCAB_SZ - 1, 0).astype(jnp.int32)
            ring_v[k, pl.ds(r * LANES, LANES)] = t2

    for k in range(NSLOT):
        build_ring_row(k, jnp.int32(k))
        pltpu.async_copy(proj_hbm.at[ring_v.at[k]], buf_v.at[k], sems[k])

    def grp_body(g, carry):
        for k in range(NSLOT):
            pltpu.make_async_copy(proj_hbm.at[pl.ds(0, BPW)], buf_v.at[k],
                                  sems[k]).wait()
            build_ring_row(k, g * NSLOT + k)
            pltpu.async_copy(proj_hbm.at[ring_v.at[k]], buf_v.at[k], sems[k],
                             add=True)
        return carry

    lax.fori_loop(1, NGRP, grp_body, 0)
    for k in range(NSLOT):
        pltpu.make_async_copy(proj_hbm.at[pl.ds(0, BPW)], buf_v.at[k],
                              sems[k]).wait()

    def comb_body(b, carry):
        for c in range(NCHUNK):
            a = bias_v[pl.ds(c * LANES, LANES)] * float(SEQ_LEN)
            for k in range(NSLOT):
                a = a + buf_v[k, b, pl.ds(c * LANES, LANES)]
            buf_v[0, b, pl.ds(c * LANES, LANES)] = a
        return carry

    lax.fori_loop(0, BPW, comb_body, 0)
    pltpu.sync_copy(buf_v.at[0], out_hbm.at[pl.ds(wid * BPW, BPW)])


def kernel(input_ids, emb_table, fc_w, fc_b):
    wt = fc_w.T
    zeros = jnp.zeros((EMBED, NCLS), jnp.float32)
    w2 = jnp.concatenate(
        [jnp.concatenate([wt, zeros], axis=1),
         jnp.concatenate([zeros, wt], axis=1)], axis=0)

    proj2 = pl.pallas_call(
        _proj_body,
        grid=(HALF // VBLK,),
        in_specs=[
            pl.BlockSpec((VBLK, EMBED), lambda i: (i, 0)),
            pl.BlockSpec((VBLK, EMBED), lambda i: (i + HALF // VBLK, 0)),
            pl.BlockSpec((2 * EMBED, 2 * NCLS), lambda i: (0, 0)),
        ],
        out_specs=pl.BlockSpec((VBLK, 2 * NCLS), lambda i: (i, 0)),
        out_shape=jax.ShapeDtypeStruct((HALF, 2 * NCLS), jnp.float32),
    )(emb_table, emb_table, w2)
    proj = proj2.reshape(VOCAB_SZ, NCLS)

    pool = pl.kernel(
        _pool_body,
        out_type=jax.ShapeDtypeStruct((BATCH_SZ, NCLS), jnp.float32),
        mesh=plsc.VectorSubcoreMesh(core_axis_name="c", subcore_axis_name="s"),
        compiler_params=pltpu.CompilerParams(use_tc_tiling_on_sc=False,
                                             needs_layout_passes=False),
        scratch_types=[
            pltpu.VMEM((BPW * SEQ_LEN,), jnp.int32),
            pltpu.VMEM((NSLOT, BPW), jnp.int32),
            pltpu.VMEM((NSLOT, BPW, NCLS), jnp.float32),
            pltpu.VMEM((NCLS,), jnp.float32),
        ] + [pltpu.SemaphoreType.DMA] * NSLOT,
    )
    return pool(input_ids.reshape(-1), proj, fc_b)


# speedup vs baseline: 1.6722x; 1.0605x over previous
import jax
import jax.numpy as jnp
from jax import lax
from jax.experimental import pallas as pl
from jax.experimental.pallas import tpu as pltpu
from jax.experimental.pallas import tpu_sc as plsc

VOCAB_SZ = 100000
EMBED = 128
NCLS = 64
BATCH_SZ = 4096
SEQ_LEN = 200
LANES = 16
NCHUNK = NCLS // LANES
NBGRP = BPW_GROUPS = 8

NC, NS = 2, 16
NW = NC * NS
BPW = BATCH_SZ // NW
NSLOT = 10
NGRP = SEQ_LEN // NSLOT

VBLK = 10000
HALF = VOCAB_SZ // 2


def _proj_body(lo_ref, hi_ref, w2_ref, out_ref):
    pair = jnp.concatenate([lo_ref[...], hi_ref[...]], axis=1)
    out_ref[...] = lax.dot_general(
        pair.astype(jnp.bfloat16), w2_ref[...].astype(jnp.bfloat16),
        dimension_numbers=(((1,), (0,)), ((), ())),
        preferred_element_type=jnp.float32)


def _pool_body(ids_hbm, proj_hbm, fcb_hbm, out_hbm, ids_v, ring_v, buf_v,
               bias_v, *sems):
    wid = lax.axis_index("s") * NC + lax.axis_index("c")
    pltpu.sync_copy(ids_hbm.at[pl.ds(wid * BPW * SEQ_LEN, BPW * SEQ_LEN)],
                    ids_v)
    pltpu.sync_copy(fcb_hbm, bias_v)

    rows = [(lax.iota(jnp.int32, LANES) + r * LANES) * SEQ_LEN
            for r in range(NBGRP)]

    def build_ring_row(k, j):
        for r in range(NBGRP):
            t = plsc.load_gather(ids_v, [rows[r] + j])
            t2 = t * 2 - jnp.where(t >= HALF, VOCAB_SZ - 1, 0).astype(jnp.int32)
            ring_v[k, pl.ds(r * LANES, LANES)] = t2

    for k in range(NSLOT):
        build_ring_row(k, jnp.int32(k))
        pltpu.async_copy(proj_hbm.at[ring_v.at[k]], buf_v.at[k], sems[k])

    def grp_body(g, carry):
        for k in range(NSLOT):
            pltpu.make_async_copy(proj_hbm.at[pl.ds(0, BPW)], buf_v.at[k],
                                  sems[k]).wait()
            build_ring_row(k, g * NSLOT + k)
            pltpu.async_copy(proj_hbm.at[ring_v.at[k]], buf_v.at[k], sems[k],
                             add=True)
        return carry

    lax.fori_loop(1, NGRP, grp_body, 0)
    for k in range(NSLOT):
        pltpu.make_async_copy(proj_hbm.at[pl.ds(0, BPW)], buf_v.at[k],
                              sems[k]).wait()

    def comb_body(b, carry):
        for c in range(NCHUNK):
            a = bias_v[pl.ds(c * LANES, LANES)] * float(SEQ_LEN)
            for k in range(NSLOT):
                a = a + buf_v[k, b, pl.ds(c * LANES, LANES)]
            buf_v[0, b, pl.ds(c * LANES, LANES)] = a
        return carry

    lax.fori_loop(0, BPW, comb_body, 0)
    pltpu.sync_copy(buf_v.at[0], out_hbm.at[pl.ds(wid * BPW, BPW)])


def kernel(input_ids, emb_table, fc_w, fc_b):
    wt = fc_w.T
    zeros = jnp.zeros((EMBED, NCLS), jnp.float32)
    w2 = jnp.concatenate(
        [jnp.concatenate([wt, zeros], axis=1),
         jnp.concatenate([zeros, wt], axis=1)], axis=0)

    proj2 = pl.pallas_call(
        _proj_body,
        grid=(HALF // VBLK,),
        in_specs=[
            pl.BlockSpec((VBLK, EMBED), lambda i: (i, 0)),
            pl.BlockSpec((VBLK, EMBED), lambda i: (i + HALF // VBLK, 0)),
            pl.BlockSpec((2 * EMBED, 2 * NCLS), lambda i: (0, 0)),
        ],
        out_specs=pl.BlockSpec((VBLK, 2 * NCLS), lambda i: (i, 0)),
        out_shape=jax.ShapeDtypeStruct((HALF, 2 * NCLS), jnp.float32),
    )(emb_table, emb_table, w2)
    proj = proj2.reshape(VOCAB_SZ, NCLS)

    pool = pl.kernel(
        _pool_body,
        out_type=jax.ShapeDtypeStruct((BATCH_SZ, NCLS), jnp.float32),
        mesh=plsc.VectorSubcoreMesh(core_axis_name="c", subcore_axis_name="s"),
        compiler_params=pltpu.CompilerParams(use_tc_tiling_on_sc=False,
                                             needs_layout_passes=False),
        scratch_types=[
            pltpu.VMEM((BPW * SEQ_LEN,), jnp.int32),
            pltpu.VMEM((NSLOT, BPW), jnp.int32),
            pltpu.VMEM((NSLOT, BPW, NCLS), jnp.float32),
            pltpu.VMEM((NCLS,), jnp.float32),
        ] + [pltpu.SemaphoreType.DMA] * NSLOT,
    )
    return pool(input_ids.reshape(-1), proj, fc_b)
